# baseline (speedup 1.0000x reference)
import jax
import jax.numpy as jnp
from jax import lax
from jax.experimental import pallas as pl
from jax.experimental.pallas import tpu as pltpu
from jax.experimental.pallas import tpu_sc as plsc

N_TOK = 32768
N_SENT = 2048
N_DOC = 16
D = 128

NC = 2
NS = 16
NW = NC * NS
CHUNK = 64
TOK_PER_W = N_TOK // NW
NCHUNK = TOK_PER_W // CHUNK
ROWS_PER_TILE = N_SENT // NS
CNT_R = N_SENT // D


NBUF = 8


def _sc_body(tok_hbm, seg_hbm, table_hbm,
             psum_hbm, pcnt_hbm,
             acc, tok_v, seg_v, bufs, cnt_loc, gsems, ssems):
    cid = lax.axis_index("c")
    sid = lax.axis_index("s")
    w = cid * NS + sid

    row0 = sid * ROWS_PER_TILE
    pltpu.sync_copy(tok_hbm.at[w], tok_v)
    pltpu.sync_copy(seg_hbm.at[w], seg_v)

    gath = [None] * NCHUNK
    scat = [None] * NCHUNK
    for j in range(NBUF - 1):
        gath[j] = pltpu.async_copy(table_hbm.at[tok_v.at[j]],
                                   bufs[j % NBUF], gsems[j % NBUF])

    zero16 = jnp.zeros((16,), jnp.float32)

    def _zero_body(r, carry):
        for c in range(D // 16):
            cnt_loc[r, pl.ds(c * 16, 16)] = zero16
        return carry

    lax.fori_loop(0, CNT_R, _zero_body, 0, unroll=False)
    for k in range(ROWS_PER_TILE // CNT_R):
        pltpu.sync_copy(cnt_loc, acc.at[pl.ds(row0 + k * CNT_R, CNT_R)])

    ones16 = jnp.ones((16,), jnp.float32)

    def _cnt_body(j, carry):
        for c in range(CHUNK // 16):
            seg = seg_v[j, pl.ds(c * 16, 16)]
            plsc.addupdate_scatter(
                cnt_loc,
                [lax.shift_right_logical(seg, 7), lax.bitwise_and(seg, 127)],
                ones16)
        return carry

    lax.fori_loop(0, NCHUNK, _cnt_body, 0, unroll=False)
    pltpu.sync_copy(cnt_loc, pcnt_hbm.at[w])
    plsc.subcore_barrier()

    for j in range(NCHUNK):
        nj = j + NBUF - 1
        if nj < NCHUNK:
            if j - 1 >= 0:
                scat[j - 1].wait()
            gath[nj] = pltpu.async_copy(table_hbm.at[tok_v.at[nj]],
                                        bufs[nj % NBUF], gsems[nj % NBUF])
        gath[j].wait()
        scat[j] = pltpu.async_copy(bufs[j % NBUF], acc.at[seg_v.at[j]],
                                   ssems[j % NBUF], add=True)
    for j in range(NCHUNK):
        if scat[j] is not None and j >= NCHUNK - NBUF:
            scat[j].wait()
    plsc.subcore_barrier()

    pltpu.sync_copy(acc.at[pl.ds(row0, ROWS_PER_TILE)],
                    psum_hbm.at[cid, pl.ds(row0, ROWS_PER_TILE)])


_sc_segsum = pl.kernel(
    _sc_body,
    out_type=(jax.ShapeDtypeStruct((NC, N_SENT, D), jnp.float32),
              jax.ShapeDtypeStruct((NW, CNT_R, D), jnp.float32)),
    mesh=plsc.VectorSubcoreMesh(core_axis_name="c", subcore_axis_name="s",
                                num_cores=NC, num_subcores=NS),
    scratch_types=[
        pltpu.VMEM_SHARED((N_SENT, D), jnp.float32),
        pltpu.VMEM((NCHUNK, CHUNK), jnp.int32),
        pltpu.VMEM((NCHUNK, CHUNK), jnp.int32),
        [pltpu.VMEM((CHUNK, D), jnp.float32) for _ in range(NBUF)],
        pltpu.VMEM((CNT_R, D), jnp.float32),
        [pltpu.SemaphoreType.DMA for _ in range(NBUF)],
        [pltpu.SemaphoreType.DMA for _ in range(NBUF)],
    ],
    compiler_params=pltpu.CompilerParams(needs_layout_passes=False),
)


def _tc_body(ps_ref, pc_ref, did_ref, ws_ref, wd_ref, out_ref):
    sent_sum = ps_ref[0] + ps_ref[1]
    cnt2d = jnp.sum(pc_ref[...], axis=0)
    cnt3 = jnp.maximum(cnt2d, 1.0)[:, :, None]
    sent_avg = (sent_sum.reshape(CNT_R, D, D) / cnt3).reshape(N_SENT, D)
    sent_embs = jnp.tanh(
        jnp.dot(sent_avg, ws_ref[...], preferred_element_type=jnp.float32))
    did = did_ref[...]
    onehot = (lax.broadcasted_iota(jnp.int32, (N_DOC, N_SENT), 0)
              == did).astype(jnp.float32)
    doc_sum = jnp.dot(onehot, sent_embs, preferred_element_type=jnp.float32)
    doc_cnt = jnp.sum(onehot, axis=1, keepdims=True)
    doc_avg = doc_sum / jnp.maximum(doc_cnt, 1.0)
    out_ref[...] = jnp.tanh(
        jnp.dot(doc_avg, wd_ref[...], preferred_element_type=jnp.float32))


_tc_finish = pl.pallas_call(
    _tc_body,
    out_shape=jax.ShapeDtypeStruct((N_DOC, D), jnp.float32),
)


def kernel(token_ids, sent_seg_ids, doc_seg_ids, table, W_sent, W_doc):
    tok3 = token_ids.astype(jnp.int32).reshape(NW, NCHUNK, CHUNK)
    seg3 = sent_seg_ids.astype(jnp.int32).reshape(NW, NCHUNK, CHUNK)
    psum, pcnt = _sc_segsum(tok3, seg3, table)
    did = doc_seg_ids.astype(jnp.int32).reshape(1, N_SENT)
    return _tc_finish(psum, pcnt, did, W_sent, W_doc)


# speedup vs baseline: 1.0177x; 1.0177x over previous
import jax
import jax.numpy as jnp
from jax import lax
from jax.experimental import pallas as pl
from jax.experimental.pallas import tpu as pltpu
from jax.experimental.pallas import tpu_sc as plsc

N_TOK = 32768
N_SENT = 2048
N_DOC = 16
D = 128

NC = 2
NS = 16
NW = NC * NS
CHUNK = 128
TOK_PER_W = N_TOK // NW
NCHUNK = TOK_PER_W // CHUNK
ROWS_PER_TILE = N_SENT // NS
CNT_R = N_SENT // D


NBUF = 4


def _sc_body(tok_hbm, seg_hbm, table_hbm,
             psum_hbm, pcnt_hbm,
             acc, tok_v, seg_v, bufs, cnt_loc, gsems, ssems):
    cid = lax.axis_index("c")
    sid = lax.axis_index("s")
    w = cid * NS + sid

    row0 = sid * ROWS_PER_TILE
    pltpu.sync_copy(tok_hbm.at[w], tok_v)

    gath = [None] * NCHUNK
    scat = [None] * NCHUNK
    for j in range(NBUF - 1):
        gath[j] = pltpu.async_copy(table_hbm.at[tok_v.at[j]],
                                   bufs[j % NBUF], gsems[j % NBUF])

    pltpu.sync_copy(seg_hbm.at[w], seg_v)

    zero16 = jnp.zeros((16,), jnp.float32)

    def _zero_body(r, carry):
        for c in range(D // 16):
            cnt_loc[r, pl.ds(c * 16, 16)] = zero16
        return carry

    lax.fori_loop(0, CNT_R, _zero_body, 0, unroll=False)
    for k in range(ROWS_PER_TILE // CNT_R):
        pltpu.sync_copy(cnt_loc, acc.at[pl.ds(row0 + k * CNT_R, CNT_R)])

    ones16 = jnp.ones((16,), jnp.float32)

    def _cnt_body(j, carry):
        for c in range(CHUNK // 16):
            seg = seg_v[j, pl.ds(c * 16, 16)]
            plsc.addupdate_scatter(
                cnt_loc,
                [lax.shift_right_logical(seg, 7), lax.bitwise_and(seg, 127)],
                ones16)
        return carry

    lax.fori_loop(0, NCHUNK, _cnt_body, 0, unroll=False)
    pltpu.sync_copy(cnt_loc, pcnt_hbm.at[w])
    plsc.subcore_barrier()

    for j in range(NCHUNK):
        nj = j + NBUF - 1
        if nj < NCHUNK:
            if j - 1 >= 0:
                scat[j - 1].wait()
            gath[nj] = pltpu.async_copy(table_hbm.at[tok_v.at[nj]],
                                        bufs[nj % NBUF], gsems[nj % NBUF])
        gath[j].wait()
        scat[j] = pltpu.async_copy(bufs[j % NBUF], acc.at[seg_v.at[j]],
                                   ssems[j % NBUF], add=True)
    for j in range(NCHUNK):
        if scat[j] is not None and j >= NCHUNK - NBUF:
            scat[j].wait()
    plsc.subcore_barrier()

    pltpu.sync_copy(acc.at[pl.ds(row0, ROWS_PER_TILE)],
                    psum_hbm.at[cid, pl.ds(row0, ROWS_PER_TILE)])


_sc_segsum = pl.kernel(
    _sc_body,
    out_type=(jax.ShapeDtypeStruct((NC, N_SENT, D), jnp.float32),
              jax.ShapeDtypeStruct((NW, CNT_R, D), jnp.float32)),
    mesh=plsc.VectorSubcoreMesh(core_axis_name="c", subcore_axis_name="s",
                                num_cores=NC, num_subcores=NS),
    scratch_types=[
        pltpu.VMEM_SHARED((N_SENT, D), jnp.float32),
        pltpu.VMEM((NCHUNK, CHUNK), jnp.int32),
        pltpu.VMEM((NCHUNK, CHUNK), jnp.int32),
        [pltpu.VMEM((CHUNK, D), jnp.float32) for _ in range(NBUF)],
        pltpu.VMEM((CNT_R, D), jnp.float32),
        [pltpu.SemaphoreType.DMA for _ in range(NBUF)],
        [pltpu.SemaphoreType.DMA for _ in range(NBUF)],
    ],
    compiler_params=pltpu.CompilerParams(needs_layout_passes=False),
)


def _tc_body(ps_ref, pc_ref, did_ref, ws_ref, wd_ref, out_ref):
    sent_sum = ps_ref[0] + ps_ref[1]
    cnt2d = jnp.sum(pc_ref[...], axis=0)
    cnt3 = jnp.maximum(cnt2d, 1.0)[:, :, None]
    sent_avg = (sent_sum.reshape(CNT_R, D, D) / cnt3).reshape(N_SENT, D)
    sent_embs = jnp.tanh(
        jnp.dot(sent_avg, ws_ref[...], preferred_element_type=jnp.float32))
    did = did_ref[...]
    onehot = (lax.broadcasted_iota(jnp.int32, (N_DOC, N_SENT), 0)
              == did).astype(jnp.float32)
    doc_sum = jnp.dot(onehot, sent_embs, preferred_element_type=jnp.float32)
    doc_cnt = jnp.sum(onehot, axis=1, keepdims=True)
    doc_avg = doc_sum / jnp.maximum(doc_cnt, 1.0)
    out_ref[...] = jnp.tanh(
        jnp.dot(doc_avg, wd_ref[...], preferred_element_type=jnp.float32))


_tc_finish = pl.pallas_call(
    _tc_body,
    out_shape=jax.ShapeDtypeStruct((N_DOC, D), jnp.float32),
)


def kernel(token_ids, sent_seg_ids, doc_seg_ids, table, W_sent, W_doc):
    tok3 = token_ids.astype(jnp.int32).reshape(NW, NCHUNK, CHUNK)
    seg3 = sent_seg_ids.astype(jnp.int32).reshape(NW, NCHUNK, CHUNK)
    psum, pcnt = _sc_segsum(tok3, seg3, table)
    did = doc_seg_ids.astype(jnp.int32).reshape(1, N_SENT)
    return _tc_finish(psum, pcnt, did, W_sent, W_doc)
